# baseline (speedup 1.0000x reference)
import functools

import jax
import jax.numpy as jnp
from jax import lax
from jax.experimental import pallas as pl
from jax.experimental.pallas import tpu as pltpu
from jax.experimental.pallas import tpu_sc as plsc

_LANES = 16
_HW = 24


def _sc_quantize(n_slabs):
    n_workers = 32
    slabs_per_worker = n_slabs // n_workers
    spr = 6
    n_rounds = slabs_per_worker // spr
    rows_per_round = spr * _HW
    mesh = plsc.VectorSubcoreMesh(core_axis_name="c", subcore_axis_name="s")

    out_f32 = jax.ShapeDtypeStruct((n_slabs, _HW, _HW), jnp.float32)
    out_i32 = jax.ShapeDtypeStruct((n_slabs, _HW, _HW), jnp.int32)

    @functools.partial(
        pl.kernel,
        out_type=[out_f32, out_f32, out_i32],
        mesh=mesh,
        compiler_params=pltpu.CompilerParams(
            needs_layout_passes=False, use_tc_tiling_on_sc=True
        ),
        scratch_types=[
            pltpu.VMEM((spr, _HW, _HW), jnp.float32),
            pltpu.VMEM((spr, _HW, _HW), jnp.float32),
            pltpu.VMEM((spr, _HW, _HW), jnp.float32),
            pltpu.VMEM((spr, _HW, _HW), jnp.float32),
            pltpu.VMEM((spr, _HW, _HW), jnp.int32),
            pltpu.VMEM((spr, _HW, _HW), jnp.int32),
            pltpu.VMEM((64,), jnp.float32),
            pltpu.VMEM((64,), jnp.float32),
            pltpu.SemaphoreType.DMA,
            pltpu.SemaphoreType.DMA,
            pltpu.SemaphoreType.DMA,
            pltpu.SemaphoreType.DMA,
        ],
    )
    def body(x_hbm, mids_hbm, cent_hbm, ste_hbm, hard_hbm, idx_hbm,
             x0, x1, h0, h1, i0, i1, mids_v, cent_v,
             sem_in0, sem_in1, sem_out0, sem_out1):
        x_b, h_b, i_b = (x0, x1), (h0, h1), (i0, i1)
        sem_in, sem_out = (sem_in0, sem_in1), (sem_out0, sem_out1)
        wid = lax.axis_index("s") * 2 + lax.axis_index("c")
        s_base = wid * slabs_per_worker
        pltpu.sync_copy(mids_hbm, mids_v)
        pltpu.sync_copy(cent_hbm, cent_v)

        def splat(k):
            return plsc.load_gather(mids_v, [jnp.full((_LANES,), k, jnp.int32)])

        m31 = splat(31)
        m15, m47 = splat(15), splat(47)
        m7, m23, m39, m55 = splat(7), splat(23), splat(39), splat(55)

        def search(xv):
            b32 = xv > m31
            t16 = jnp.where(b32, m47, m15)
            b16 = xv > t16
            ta = jnp.where(b32, m39, m7)
            tb = jnp.where(b32, m55, m23)
            b8 = xv > jnp.where(b16, tb, ta)
            pos = (jnp.where(b32, jnp.int32(32), jnp.int32(0))
                   + jnp.where(b16, jnp.int32(16), jnp.int32(0))
                   + jnp.where(b8, jnp.int32(8), jnp.int32(0)))
            for st in (4, 2, 1):
                m = plsc.load_gather(mids_v, [pos + (st - 1)])
                pos = pos + jnp.where(xv > m, jnp.int32(st), jnp.int32(0))
            return pos, plsc.load_gather(cent_v, [pos])

        def slab_slice(r):
            return pl.ds(s_base + r * spr, spr)

        h_in = [None, None]
        h_out = [None] * n_rounds
        h_in[0] = pltpu.async_copy(x_hbm.at[slab_slice(0)], x_b[0], sem_in[0])
        for r in range(n_rounds):
            b = r % 2
            if r + 1 < n_rounds:
                h_in[1 - b] = pltpu.async_copy(
                    x_hbm.at[slab_slice(r + 1)], x_b[1 - b], sem_in[1 - b])
            h_in[b].wait()
            if r >= 2:
                for h in h_out[r - 2]:
                    h.wait()
            x_v, hard_v, idx_v = x_b[b], h_b[b], i_b[b]

            @plsc.parallel_loop(0, rows_per_round, 1, unroll=4)
            def row(i):
                s = i // _HW
                rr = i % _HW
                for off in (0, 8):
                    xv = x_v[s, rr, pl.ds(off, _LANES)]
                    pos, hard = search(xv)
                    hard_v[s, rr, pl.ds(off, _LANES)] = hard
                    idx_v[s, rr, pl.ds(off, _LANES)] = pos

            sl = slab_slice(r)
            h_out[r] = [
                pltpu.async_copy(hard_v, ste_hbm.at[sl], sem_out[b]),
                pltpu.async_copy(hard_v, hard_hbm.at[sl], sem_out[b]),
                pltpu.async_copy(idx_v, idx_hbm.at[sl], sem_out[b]),
            ]
        for r in (n_rounds - 2, n_rounds - 1):
            for h in h_out[r]:
                h.wait()

    return body


def kernel(x, centers):
    n, c, h, w = x.shape
    mids = jnp.concatenate(
        [(centers[:-1] + centers[1:]) * 0.5, jnp.full((1,), jnp.inf, jnp.float32)]
    )
    x3 = x.reshape(n * c, h, w)
    ste, hard, idx = _sc_quantize(n * c)(x3, mids, centers)
    shape = (n, c, h, w)
    return (ste.reshape(shape), hard.reshape(shape), idx.reshape(shape))


# speedup vs baseline: 2.6393x; 2.6393x over previous
import functools

import jax
import jax.numpy as jnp
from jax import lax
from jax.experimental import pallas as pl
from jax.experimental.pallas import tpu as pltpu
from jax.experimental.pallas import tpu_sc as plsc

_LANES = 16


def _sc_quantize(n, hh, ww, c):
    n_workers = 32
    n_slabs = n * hh
    slabs_per_worker = n_slabs // n_workers
    workers_per_n = hh // slabs_per_worker
    vecs_per_row = c // _LANES
    vecs_per_slab = ww * vecs_per_row
    mesh = plsc.VectorSubcoreMesh(core_axis_name="c", subcore_axis_name="s")

    out_f32 = jax.ShapeDtypeStruct((n, hh, ww, c), jnp.float32)
    out_i32 = jax.ShapeDtypeStruct((n, hh, ww, c), jnp.int32)

    @functools.partial(
        pl.kernel,
        out_type=[out_f32, out_f32, out_i32],
        mesh=mesh,
        compiler_params=pltpu.CompilerParams(
            needs_layout_passes=False, use_tc_tiling_on_sc=True
        ),
        scratch_types=[
            pltpu.VMEM((ww, c), jnp.float32),
            pltpu.VMEM((ww, c), jnp.float32),
            pltpu.VMEM((ww, c), jnp.float32),
            pltpu.VMEM((ww, c), jnp.float32),
            pltpu.VMEM((ww, c), jnp.int32),
            pltpu.VMEM((ww, c), jnp.int32),
            pltpu.VMEM((64,), jnp.float32),
            pltpu.VMEM((64,), jnp.float32),
            pltpu.SemaphoreType.DMA,
            pltpu.SemaphoreType.DMA,
            pltpu.SemaphoreType.DMA,
            pltpu.SemaphoreType.DMA,
        ],
    )
    def body(x_hbm, mids_hbm, cent_hbm, ste_hbm, hard_hbm, idx_hbm,
             x0, x1, h0, h1, i0, i1, mids_v, cent_v,
             sem_in0, sem_in1, sem_out0, sem_out1):
        x_b, h_b, i_b = (x0, x1), (h0, h1), (i0, i1)
        sem_in, sem_out = (sem_in0, sem_in1), (sem_out0, sem_out1)
        wid = lax.axis_index("s") * 2 + lax.axis_index("c")
        n0 = wid // workers_per_n
        h_base = (wid % workers_per_n) * slabs_per_worker
        pltpu.sync_copy(mids_hbm, mids_v)
        pltpu.sync_copy(cent_hbm, cent_v)

        def splat(k):
            return plsc.load_gather(mids_v, [jnp.full((_LANES,), k, jnp.int32)])

        m31 = splat(31)
        m15, m47 = splat(15), splat(47)
        m7, m23, m39, m55 = splat(7), splat(23), splat(39), splat(55)

        def search(xv):
            b32 = xv > m31
            t16 = jnp.where(b32, m47, m15)
            b16 = xv > t16
            ta = jnp.where(b32, m39, m7)
            tb = jnp.where(b32, m55, m23)
            b8 = xv > jnp.where(b16, tb, ta)
            pos = (jnp.where(b32, jnp.int32(32), jnp.int32(0))
                   + jnp.where(b16, jnp.int32(16), jnp.int32(0))
                   + jnp.where(b8, jnp.int32(8), jnp.int32(0)))
            for st in (4, 2, 1):
                m = plsc.load_gather(mids_v, [pos + (st - 1)])
                pos = pos + jnp.where(xv > m, jnp.int32(st), jnp.int32(0))
            return pos, plsc.load_gather(cent_v, [pos])

        h_in = [None, None]
        h_out = [None] * slabs_per_worker
        h_in[0] = pltpu.async_copy(x_hbm.at[n0, h_base], x_b[0], sem_in[0])
        for r in range(slabs_per_worker):
            b = r % 2
            if r + 1 < slabs_per_worker:
                h_in[1 - b] = pltpu.async_copy(
                    x_hbm.at[n0, h_base + r + 1], x_b[1 - b], sem_in[1 - b])
            h_in[b].wait()
            if r >= 2:
                for h in h_out[r - 2]:
                    h.wait()
            x_v, hard_v, idx_v = x_b[b], h_b[b], i_b[b]

            @plsc.parallel_loop(0, vecs_per_slab, 1, unroll=4)
            def vec(i):
                w = i // vecs_per_row
                off = (i % vecs_per_row) * _LANES
                xv = x_v[w, pl.ds(off, _LANES)]
                pos, hard = search(xv)
                hard_v[w, pl.ds(off, _LANES)] = hard
                idx_v[w, pl.ds(off, _LANES)] = pos

            h_out[r] = [
                pltpu.async_copy(hard_v, ste_hbm.at[n0, h_base + r], sem_out[b]),
                pltpu.async_copy(hard_v, hard_hbm.at[n0, h_base + r], sem_out[b]),
                pltpu.async_copy(idx_v, idx_hbm.at[n0, h_base + r], sem_out[b]),
            ]
        for r in (slabs_per_worker - 2, slabs_per_worker - 1):
            for h in h_out[r]:
                h.wait()

    return body


def kernel(x, centers):
    n, c, h, w = x.shape
    mids = jnp.concatenate(
        [(centers[:-1] + centers[1:]) * 0.5, jnp.full((1,), jnp.inf, jnp.float32)]
    )
    xt = jnp.transpose(x, (0, 2, 3, 1))
    ste, hard, idx = _sc_quantize(n, h, w, c)(xt, mids, centers)
    back = (0, 3, 1, 2)
    return (jnp.transpose(ste, back), jnp.transpose(hard, back),
            jnp.transpose(idx, back))
